# baseline (speedup 1.0000x reference)
import math

import jax
import jax.numpy as jnp
import numpy as np
from jax import lax
from jax.experimental import pallas as pl
from jax.experimental.pallas import tpu as pltpu
from jax.experimental.pallas import tpu_sc as plsc

D_MODEL = 128
MAX_LEN = 1500
VOCAB = 28
BATCH = 256
SEQ = 1024

NC, NS, LANES = 2, 16, 16
NW = NC * NS
TOK_PER_W = BATCH * SEQ // NW
QUARTERS = 4
Q = SEQ // QUARTERS
QD = Q * D_MODEL
SEQ_PER_W = TOK_PER_W // SEQ


def _make_pe_np(max_len, d_model):
    position = np.arange(0, max_len, dtype=np.float32)[:, None]
    div_term = np.exp(
        np.arange(0, d_model, 2).astype(np.float32) * -(math.log(10000.0) / d_model)
    )
    pe = np.zeros((max_len, d_model), dtype=np.float32)
    pe[:, 0::2] = np.sin(position * div_term)
    pe[:, 1::2] = np.cos(position * div_term)
    return pe


_PE_NP = _make_pe_np(MAX_LEN, D_MODEL)[:SEQ].reshape(-1)


def _sc_embed(tokens_flat, table_flat, pe_flat):
    mesh = plsc.VectorSubcoreMesh(
        core_axis_name="c", subcore_axis_name="s", num_cores=NC, num_subcores=NS
    )

    def body(tok_hbm, table_hbm, pe_hbm, out_hbm,
             table_v, pe_v, idx_v, rows0, rows1, sem0, sem1):
        wid = lax.axis_index("s") * NC + lax.axis_index("c")
        base = wid * TOK_PER_W
        pltpu.sync_copy(table_hbm, table_v)
        lanes128 = lax.broadcasted_iota(jnp.int32, (LANES,), 0) * D_MODEL
        rows = (rows0, rows1)
        sems = (sem0, sem1)

        def compute_chunk(g, rows_b):
            pltpu.sync_copy(tok_hbm.at[pl.ds(g, Q)], idx_v)

            def rb_body(rb, _):
                tokv = idx_v[pl.ds(rb * LANES, LANES)] * D_MODEL
                sbase = rb * (LANES * D_MODEL)

                def c4_body(c4, _):
                    for dc in range(4):
                        cc = c4 * 4 + dc
                        tv = plsc.load_gather(table_v, [tokv + cc])
                        si = lanes128 + (sbase + cc)
                        pv = plsc.load_gather(pe_v, [si])
                        plsc.store_scatter(rows_b, [si], tv + pv)
                    return 0

                lax.fori_loop(0, D_MODEL // 4, c4_body, 0)
                return 0

            lax.fori_loop(0, Q // LANES, rb_body, 0)

        for q in range(QUARTERS):
            pltpu.sync_copy(pe_hbm.at[pl.ds(q * QD, QD)], pe_v)

            def s2_body(s2, _, q=q):
                for b in range(2):
                    s = s2 * 2 + b
                    g = base + s * SEQ + q * Q

                    @pl.when(s2 > 0)
                    def _wait(b=b, g=g):
                        pltpu.make_async_copy(
                            rows[b], out_hbm.at[pl.ds(g * D_MODEL, QD)], sems[b]
                        ).wait()

                    compute_chunk(g, rows[b])
                    pltpu.async_copy(
                        rows[b], out_hbm.at[pl.ds(g * D_MODEL, QD)], sems[b]
                    )
                return 0

            lax.fori_loop(0, SEQ_PER_W // 2, s2_body, 0)
            for b in range(2):
                pltpu.make_async_copy(
                    rows[b], out_hbm.at[pl.ds(base * D_MODEL, QD)], sems[b]
                ).wait()

    run = pl.kernel(
        body,
        out_type=jax.ShapeDtypeStruct((BATCH * SEQ * D_MODEL,), jnp.float32),
        mesh=mesh,
        compiler_params=pltpu.CompilerParams(needs_layout_passes=False),
        scratch_types=[
            pltpu.VMEM((VOCAB * D_MODEL,), jnp.float32),
            pltpu.VMEM((QD,), jnp.float32),
            pltpu.VMEM((Q,), jnp.int32),
            pltpu.VMEM((QD,), jnp.float32),
            pltpu.VMEM((QD,), jnp.float32),
            pltpu.SemaphoreType.DMA,
            pltpu.SemaphoreType.DMA,
        ],
    )
    return run(tokens_flat, table_flat, pe_flat)


def kernel(tokens, table):
    tokens_flat = tokens.reshape(-1).astype(jnp.int32)
    out = _sc_embed(tokens_flat, table.reshape(-1), jnp.asarray(_PE_NP))
    return out.reshape(BATCH, SEQ, D_MODEL)


# speedup vs baseline: 2.0163x; 2.0163x over previous
import math

import jax
import jax.numpy as jnp
import numpy as np
from jax import lax
from jax.experimental import pallas as pl
from jax.experimental.pallas import tpu as pltpu
from jax.experimental.pallas import tpu_sc as plsc

D_MODEL = 128
MAX_LEN = 1500
VOCAB = 28
BATCH = 256
SEQ = 1024

NC, NS, LANES = 2, 16, 16
NW = NC * NS
TOK_PER_W = BATCH * SEQ // NW
QUARTERS = 4
Q = SEQ // QUARTERS
QD = Q * D_MODEL
SEQ_PER_W = TOK_PER_W // SEQ


def _make_pe_np(max_len, d_model):
    position = np.arange(0, max_len, dtype=np.float32)[:, None]
    div_term = np.exp(
        np.arange(0, d_model, 2).astype(np.float32) * -(math.log(10000.0) / d_model)
    )
    pe = np.zeros((max_len, d_model), dtype=np.float32)
    pe[:, 0::2] = np.sin(position * div_term)
    pe[:, 1::2] = np.cos(position * div_term)
    return pe


_PE_NP = _make_pe_np(MAX_LEN, D_MODEL)[:SEQ].reshape(-1)


def _sc_embed(tokens_flat, table_flat, pe_flat):
    mesh = plsc.VectorSubcoreMesh(
        core_axis_name="c", subcore_axis_name="s", num_cores=NC, num_subcores=NS
    )

    def body(tok_hbm, table_hbm, pe_hbm, out_hbm,
             table_v, pe_v, idx_v, rows0, rows1, sem0, sem1):
        wid = lax.axis_index("s") * NC + lax.axis_index("c")
        base = wid * TOK_PER_W
        pltpu.sync_copy(table_hbm, table_v)
        lanes128 = lax.broadcasted_iota(jnp.int32, (LANES,), 0) * D_MODEL
        rows = (rows0, rows1)
        sems = (sem0, sem1)

        def compute_chunk(g, rows_b):
            pltpu.sync_copy(tok_hbm.at[pl.ds(g, Q)], idx_v)

            def rb_body(rb, _):
                tokv = idx_v[pl.ds(rb * LANES, LANES)] * D_MODEL
                sbase = rb * (LANES * D_MODEL)

                @plsc.parallel_loop(0, D_MODEL, unroll=8)
                def _c_body(c):
                    tv = plsc.load_gather(table_v, [tokv + c])
                    si = lanes128 + (sbase + c)
                    pv = plsc.load_gather(pe_v, [si])
                    plsc.store_scatter(rows_b, [si], tv + pv)

                return 0

            lax.fori_loop(0, Q // LANES, rb_body, 0)

        def q_body(q, _):
            pltpu.sync_copy(pe_hbm.at[pl.ds(q * QD, QD)], pe_v)

            def s2_body(s2, _):
                for b in range(2):
                    s = s2 * 2 + b
                    g = base + s * SEQ + q * Q

                    @pl.when(jnp.logical_or(q > 0, s2 > 0))
                    def _wait(b=b):
                        pltpu.make_async_copy(
                            rows[b], out_hbm.at[pl.ds(0, QD)], sems[b]
                        ).wait()

                    compute_chunk(g, rows[b])
                    pltpu.async_copy(
                        rows[b], out_hbm.at[pl.ds(g * D_MODEL, QD)], sems[b]
                    )
                return 0

            lax.fori_loop(0, SEQ_PER_W // 2, s2_body, 0)
            return 0

        lax.fori_loop(0, QUARTERS, q_body, 0)
        for b in range(2):
            pltpu.make_async_copy(
                rows[b], out_hbm.at[pl.ds(0, QD)], sems[b]
            ).wait()

    run = pl.kernel(
        body,
        out_type=jax.ShapeDtypeStruct((BATCH * SEQ * D_MODEL,), jnp.float32),
        mesh=mesh,
        compiler_params=pltpu.CompilerParams(needs_layout_passes=False),
        scratch_types=[
            pltpu.VMEM((VOCAB * D_MODEL,), jnp.float32),
            pltpu.VMEM((QD,), jnp.float32),
            pltpu.VMEM((Q,), jnp.int32),
            pltpu.VMEM((QD,), jnp.float32),
            pltpu.VMEM((QD,), jnp.float32),
            pltpu.SemaphoreType.DMA,
            pltpu.SemaphoreType.DMA,
        ],
    )
    return run(tokens_flat, table_flat, pe_flat)


def kernel(tokens, table):
    tokens_flat = tokens.reshape(-1).astype(jnp.int32)
    out = _sc_embed(tokens_flat, table.reshape(-1), jnp.asarray(_PE_NP))
    return out.reshape(BATCH, SEQ, D_MODEL)


# speedup vs baseline: 9.8850x; 4.9027x over previous
import math

import jax
import jax.numpy as jnp
import numpy as np
from jax import lax
from jax.experimental import pallas as pl
from jax.experimental.pallas import tpu as pltpu
from jax.experimental.pallas import tpu_sc as plsc

D_MODEL = 128
MAX_LEN = 1500
VOCAB = 28
BATCH = 256
SEQ = 1024

NC, NS, LANES = 2, 16, 16
NW = NC * NS
TOK_PER_W = BATCH * SEQ // NW
QUARTERS = 4
Q = SEQ // QUARTERS
QD = Q * D_MODEL
SEQ_PER_W = TOK_PER_W // SEQ


def _make_pe_np(max_len, d_model):
    position = np.arange(0, max_len, dtype=np.float32)[:, None]
    div_term = np.exp(
        np.arange(0, d_model, 2).astype(np.float32) * -(math.log(10000.0) / d_model)
    )
    pe = np.zeros((max_len, d_model), dtype=np.float32)
    pe[:, 0::2] = np.sin(position * div_term)
    pe[:, 1::2] = np.cos(position * div_term)
    return pe


_PE_NP = _make_pe_np(MAX_LEN, D_MODEL)[:SEQ].reshape(-1)


def _sc_embed(tokens_flat, table_flat, pe_flat):
    mesh = plsc.VectorSubcoreMesh(
        core_axis_name="c", subcore_axis_name="s", num_cores=NC, num_subcores=NS
    )

    def body(tok_hbm, table_hbm, pe_hbm, out_hbm,
             table_v, pe_v, idx_v, rows0, rows1, sem0, sem1):
        wid = lax.axis_index("s") * NC + lax.axis_index("c")
        base = wid * TOK_PER_W
        pltpu.sync_copy(table_hbm, table_v)
        lanes128 = lax.broadcasted_iota(jnp.int32, (LANES,), 0) * D_MODEL
        rows = (rows0, rows1)
        sems = (sem0, sem1)

        def compute_chunk(g, rows_b):
            pltpu.sync_copy(tok_hbm.at[pl.ds(g, Q)], idx_v)

            @plsc.parallel_loop(0, Q // LANES, unroll=1)
            def _rb_body(rb):
                tokv = idx_v[pl.ds(rb * LANES, LANES)] * D_MODEL
                gbase = rb * (LANES * D_MODEL)
                for lane in range(LANES):
                    tbase = tokv[lane]
                    rbase = gbase + lane * D_MODEL
                    for j in range(D_MODEL // LANES):
                        tv = table_v[pl.ds(tbase + j * LANES, LANES)]
                        pv = pe_v[pl.ds(rbase + j * LANES, LANES)]
                        rows_b[pl.ds(rbase + j * LANES, LANES)] = tv + pv

        def q_body(q, _):
            pltpu.sync_copy(pe_hbm.at[pl.ds(q * QD, QD)], pe_v)

            def s2_body(s2, _):
                for b in range(2):
                    s = s2 * 2 + b
                    g = base + s * SEQ + q * Q

                    @pl.when(jnp.logical_or(q > 0, s2 > 0))
                    def _wait(b=b):
                        pltpu.make_async_copy(
                            rows[b], out_hbm.at[pl.ds(0, QD)], sems[b]
                        ).wait()

                    compute_chunk(g, rows[b])
                    pltpu.async_copy(
                        rows[b], out_hbm.at[pl.ds(g * D_MODEL, QD)], sems[b]
                    )
                return 0

            lax.fori_loop(0, SEQ_PER_W // 2, s2_body, 0)
            return 0

        lax.fori_loop(0, QUARTERS, q_body, 0)
        for b in range(2):
            pltpu.make_async_copy(
                rows[b], out_hbm.at[pl.ds(0, QD)], sems[b]
            ).wait()

    run = pl.kernel(
        body,
        out_type=jax.ShapeDtypeStruct((BATCH * SEQ * D_MODEL,), jnp.float32),
        mesh=mesh,
        compiler_params=pltpu.CompilerParams(needs_layout_passes=False),
        scratch_types=[
            pltpu.VMEM((VOCAB * D_MODEL,), jnp.float32),
            pltpu.VMEM((QD,), jnp.float32),
            pltpu.VMEM((Q,), jnp.int32),
            pltpu.VMEM((QD,), jnp.float32),
            pltpu.VMEM((QD,), jnp.float32),
            pltpu.SemaphoreType.DMA,
            pltpu.SemaphoreType.DMA,
        ],
    )
    return run(tokens_flat, table_flat, pe_flat)


def kernel(tokens, table):
    tokens_flat = tokens.reshape(-1).astype(jnp.int32)
    out = _sc_embed(tokens_flat, table.reshape(-1), jnp.asarray(_PE_NP))
    return out.reshape(BATCH, SEQ, D_MODEL)


# speedup vs baseline: 10.9821x; 1.1110x over previous
import math

import jax
import jax.numpy as jnp
import numpy as np
from jax import lax
from jax.experimental import pallas as pl
from jax.experimental.pallas import tpu as pltpu
from jax.experimental.pallas import tpu_sc as plsc

D_MODEL = 128
MAX_LEN = 1500
VOCAB = 28
BATCH = 256
SEQ = 1024

NC, NS, LANES = 2, 16, 16
NW = NC * NS
TOK_PER_W = BATCH * SEQ // NW
QUARTERS = 4
Q = SEQ // QUARTERS
QD = Q * D_MODEL
SEQ_PER_W = TOK_PER_W // SEQ


def _make_pe_np(max_len, d_model):
    position = np.arange(0, max_len, dtype=np.float32)[:, None]
    div_term = np.exp(
        np.arange(0, d_model, 2).astype(np.float32) * -(math.log(10000.0) / d_model)
    )
    pe = np.zeros((max_len, d_model), dtype=np.float32)
    pe[:, 0::2] = np.sin(position * div_term)
    pe[:, 1::2] = np.cos(position * div_term)
    return pe


_PE_NP = _make_pe_np(MAX_LEN, D_MODEL)[:SEQ].reshape(-1)


def _sc_embed(tokens_flat, table_flat, pe_flat):
    mesh = plsc.VectorSubcoreMesh(
        core_axis_name="c", subcore_axis_name="s", num_cores=NC, num_subcores=NS
    )

    def body(tok_hbm, table_hbm, pe_hbm, out_hbm,
             table_v, pe_v, idx_v, rows0, rows1, sem0, sem1):
        wid = lax.axis_index("s") * NC + lax.axis_index("c")
        base = wid * TOK_PER_W
        pltpu.sync_copy(table_hbm, table_v)
        pltpu.sync_copy(tok_hbm.at[pl.ds(base, TOK_PER_W)], idx_v)
        lanes128 = lax.broadcasted_iota(jnp.int32, (LANES,), 0) * D_MODEL
        rows = (rows0, rows1)
        sems = (sem0, sem1)

        def compute_chunk(loc, rows_b):
            @plsc.parallel_loop(0, Q // LANES, unroll=1)
            def _rb_body(rb):
                tokv = idx_v[pl.ds(loc + rb * LANES, LANES)] * D_MODEL
                gbase = rb * (LANES * D_MODEL)
                for lane in range(LANES):
                    tbase = tokv[lane]
                    rbase = gbase + lane * D_MODEL
                    for j in range(D_MODEL // LANES):
                        tv = table_v[pl.ds(tbase + j * LANES, LANES)]
                        pv = pe_v[pl.ds(rbase + j * LANES, LANES)]
                        rows_b[pl.ds(rbase + j * LANES, LANES)] = tv + pv

        def q_body(q, _):
            pltpu.sync_copy(pe_hbm.at[pl.ds(q * QD, QD)], pe_v)

            def s2_body(s2, _):
                for b in range(2):
                    s = s2 * 2 + b
                    g = base + s * SEQ + q * Q

                    @pl.when(jnp.logical_or(q > 0, s2 > 0))
                    def _wait(b=b):
                        pltpu.make_async_copy(
                            rows[b], out_hbm.at[pl.ds(0, QD)], sems[b]
                        ).wait()

                    compute_chunk(s * SEQ + q * Q, rows[b])
                    pltpu.async_copy(
                        rows[b], out_hbm.at[pl.ds(g * D_MODEL, QD)], sems[b]
                    )
                return 0

            lax.fori_loop(0, SEQ_PER_W // 2, s2_body, 0)
            return 0

        lax.fori_loop(0, QUARTERS, q_body, 0)
        for b in range(2):
            pltpu.make_async_copy(
                rows[b], out_hbm.at[pl.ds(0, QD)], sems[b]
            ).wait()

    run = pl.kernel(
        body,
        out_type=jax.ShapeDtypeStruct((BATCH * SEQ * D_MODEL,), jnp.float32),
        mesh=mesh,
        compiler_params=pltpu.CompilerParams(needs_layout_passes=False),
        scratch_types=[
            pltpu.VMEM((VOCAB * D_MODEL,), jnp.float32),
            pltpu.VMEM((QD,), jnp.float32),
            pltpu.VMEM((TOK_PER_W,), jnp.int32),
            pltpu.VMEM((QD,), jnp.float32),
            pltpu.VMEM((QD,), jnp.float32),
            pltpu.SemaphoreType.DMA,
            pltpu.SemaphoreType.DMA,
        ],
    )
    return run(tokens_flat, table_flat, pe_flat)


def kernel(tokens, table):
    tokens_flat = tokens.reshape(-1).astype(jnp.int32)
    out = _sc_embed(tokens_flat, table.reshape(-1), jnp.asarray(_PE_NP))
    return out.reshape(BATCH, SEQ, D_MODEL)
